# Initial kernel scaffold; baseline (speedup 1.0000x reference)
import jax
import jax.numpy as jnp
from jax.experimental import pallas as pl


def kernel(inputs, codebook):
    raise NotImplementedError("write your pallas kernel here")


# speedup vs baseline: 1.0200x; 1.0200x over previous
import functools

import jax
import jax.numpy as jnp
from jax import lax
from jax.experimental import pallas as pl
from jax.experimental.pallas import tpu as pltpu
from jax.experimental.pallas import tpu_sc as plsc

N_ROWS = 8192
DIM = 256
K_CODES = 8192
COMMIT = 0.25

BM = 1024
BN = 512
BM3 = 256
BBIN = 1024


def _argmin_body(x_ref, c_ref, o_ref):
    x = x_ref[...]
    x2 = jnp.sum(x * x, axis=1, keepdims=True)
    rmin = jnp.full((BM,), jnp.inf, jnp.float32)
    rarg = jnp.zeros((BM,), jnp.int32)
    for j in range(K_CODES // BN):
        c = c_ref[:, j * BN:(j + 1) * BN]
        c2 = jnp.sum(c * c, axis=0, keepdims=True)
        m = jnp.dot(x, c, preferred_element_type=jnp.float32)
        d = x2 + c2 - 2.0 * m
        bmin = jnp.min(d, axis=1, keepdims=True)
        iota = lax.broadcasted_iota(jnp.int32, (BM, BN), 1)
        cand = jnp.where(d == bmin, iota, BN)
        barg = jnp.min(cand, axis=1) + j * BN
        bminf = bmin[:, 0]
        better = bminf < rmin
        rarg = jnp.where(better, barg, rarg)
        rmin = jnp.where(better, bminf, rmin)
    o_ref[...] = rarg


def _argmin_indices(flat_x, codebook):
    grid = (N_ROWS // BM,)
    return pl.pallas_call(
        _argmin_body,
        grid=grid,
        in_specs=[
            pl.BlockSpec((BM, DIM), lambda i: (i, 0)),
            pl.BlockSpec((DIM, K_CODES), lambda i: (0, 0)),
        ],
        out_specs=pl.BlockSpec((BM,), lambda i: (i,)),
        out_shape=jax.ShapeDtypeStruct((N_ROWS,), jnp.int32),
    )(flat_x, codebook)


def _gather_quantized(codebook_t, idx):
    info = plsc.get_sparse_core_info()
    nw = info.num_cores * info.num_subcores
    b_per_w = N_ROWS // nw

    mesh = plsc.VectorSubcoreMesh(core_axis_name="c", subcore_axis_name="s")

    @functools.partial(
        pl.kernel,
        mesh=mesh,
        out_type=jax.ShapeDtypeStruct((N_ROWS, DIM), jnp.float32),
        scratch_types=[
            pltpu.VMEM((b_per_w,), jnp.int32),
            pltpu.VMEM((b_per_w, DIM), jnp.float32),
            pltpu.SemaphoreType.DMA,
        ],
    )
    def gather(table_hbm, idx_hbm, out_hbm, idx_v, rows_v, sem):
        wid = lax.axis_index("s") * info.num_cores + lax.axis_index("c")
        base = wid * b_per_w
        pltpu.sync_copy(idx_hbm.at[pl.ds(base, b_per_w)], idx_v)
        pltpu.async_copy(table_hbm.at[idx_v], rows_v, sem).wait()
        pltpu.sync_copy(rows_v, out_hbm.at[pl.ds(base, b_per_w)])

    return gather(codebook_t, idx)


def _finalize_body(x_ref, q_ref, idx_ref, ste_ref, ppl_ref, cb_ref, cm_ref,
                   loss_s, cnt_s):
    i = pl.program_id(0)
    ni = pl.num_programs(0)
    x = x_ref[...]
    q = q_ref[...]
    diff = q - x
    ste_ref[...] = x + diff
    sq = jnp.sum(diff * diff)

    @pl.when(i == 0)
    def _():
        loss_s[0] = sq
        cnt_s[...] = jnp.zeros((K_CODES,), jnp.float32)

    @pl.when(i > 0)
    def _():
        loss_s[0] = loss_s[0] + sq

    idx = idx_ref[...]
    nb = K_CODES // BBIN
    for jb in range(nb):
        bins = jb * BBIN + lax.broadcasted_iota(jnp.int32, (1, BBIN), 1)
        hits = (idx[:, None] == bins).astype(jnp.float32)
        cnt_s[pl.ds(jb * BBIN, BBIN)] = (
            cnt_s[pl.ds(jb * BBIN, BBIN)] + jnp.sum(hits, axis=0))

    @pl.when(i == ni - 1)
    def _():
        mse = loss_s[0] / jnp.float32(N_ROWS * DIM)
        cb_ref[...] = mse[None, None]
        cm_ref[...] = (COMMIT * mse)[None, None]
        p = cnt_s[...] * jnp.float32(1.0 / N_ROWS)
        entropy = -jnp.sum(p * jnp.log(p + 1e-10))
        ppl_ref[...] = jnp.exp(entropy)[None, None]


def _finalize(flat_x, quantized, idx):
    grid = (N_ROWS // BM3,)
    scalar = jax.ShapeDtypeStruct((1, 1), jnp.float32)
    return pl.pallas_call(
        _finalize_body,
        grid=grid,
        in_specs=[
            pl.BlockSpec((BM3, DIM), lambda i: (i, 0)),
            pl.BlockSpec((BM3, DIM), lambda i: (i, 0)),
            pl.BlockSpec((BM3,), lambda i: (i,)),
        ],
        out_specs=[
            pl.BlockSpec((BM3, DIM), lambda i: (i, 0)),
            pl.BlockSpec((1, 1), lambda i: (0, 0)),
            pl.BlockSpec((1, 1), lambda i: (0, 0)),
            pl.BlockSpec((1, 1), lambda i: (0, 0)),
        ],
        out_shape=[
            jax.ShapeDtypeStruct((N_ROWS, DIM), jnp.float32),
            scalar, scalar, scalar,
        ],
        scratch_shapes=[
            pltpu.SMEM((1,), jnp.float32),
            pltpu.VMEM((K_CODES,), jnp.float32),
        ],
    )(flat_x, quantized, idx)


def kernel(inputs, codebook):
    flat_x = inputs.reshape(-1, DIM)
    idx = _argmin_indices(flat_x, codebook)
    quantized = _gather_quantized(codebook.T, idx)
    ste, ppl, cb_loss, cm_loss = _finalize(flat_x, quantized, idx)
    return (
        ste.reshape(inputs.shape),
        ppl.reshape(()),
        cb_loss.reshape(()),
        cm_loss.reshape(()),
        idx.reshape(inputs.shape[0], -1),
    )


# speedup vs baseline: 1.0554x; 1.0347x over previous
import functools

import jax
import jax.numpy as jnp
from jax import lax
from jax.experimental import pallas as pl
from jax.experimental.pallas import tpu as pltpu
from jax.experimental.pallas import tpu_sc as plsc

N_ROWS = 8192
DIM = 256
K_CODES = 8192
COMMIT = 0.25

BM = 2048
BN = 512
BM3 = 256
BBIN = 1024


def _argmin_body(x_ref, c_ref, o_ref):
    x = x_ref[...]
    x2 = jnp.sum(x * x, axis=1, keepdims=True)
    rmin = jnp.full((BM,), jnp.inf, jnp.float32)
    rarg = jnp.zeros((BM,), jnp.int32)
    for j in range(K_CODES // BN):
        c = c_ref[:, j * BN:(j + 1) * BN]
        c2 = jnp.sum(c * c, axis=0, keepdims=True)
        m = jnp.dot(x, c, preferred_element_type=jnp.float32)
        d = x2 + c2 - 2.0 * m
        bmin = jnp.min(d, axis=1, keepdims=True)
        iota = lax.broadcasted_iota(jnp.int32, (BM, BN), 1)
        cand = jnp.where(d == bmin, iota, BN)
        barg = jnp.min(cand, axis=1) + j * BN
        bminf = bmin[:, 0]
        better = bminf < rmin
        rarg = jnp.where(better, barg, rarg)
        rmin = jnp.where(better, bminf, rmin)
    o_ref[...] = rarg


def _argmin_indices(flat_x, codebook):
    grid = (N_ROWS // BM,)
    return pl.pallas_call(
        _argmin_body,
        grid=grid,
        in_specs=[
            pl.BlockSpec((BM, DIM), lambda i: (i, 0)),
            pl.BlockSpec((DIM, K_CODES), lambda i: (0, 0)),
        ],
        out_specs=pl.BlockSpec((BM,), lambda i: (i,)),
        out_shape=jax.ShapeDtypeStruct((N_ROWS,), jnp.int32),
    )(flat_x, codebook)


def _gather_quantized(codebook_t, idx):
    info = plsc.get_sparse_core_info()
    nw = info.num_cores * info.num_subcores
    b_per_w = N_ROWS // nw

    mesh = plsc.VectorSubcoreMesh(core_axis_name="c", subcore_axis_name="s")

    @functools.partial(
        pl.kernel,
        mesh=mesh,
        out_type=jax.ShapeDtypeStruct((N_ROWS, DIM), jnp.float32),
        scratch_types=[
            pltpu.VMEM((b_per_w,), jnp.int32),
            pltpu.VMEM((b_per_w, DIM), jnp.float32),
            pltpu.SemaphoreType.DMA,
        ],
    )
    def gather(table_hbm, idx_hbm, out_hbm, idx_v, rows_v, sem):
        wid = lax.axis_index("s") * info.num_cores + lax.axis_index("c")
        base = wid * b_per_w
        pltpu.sync_copy(idx_hbm.at[pl.ds(base, b_per_w)], idx_v)
        pltpu.async_copy(table_hbm.at[idx_v], rows_v, sem).wait()
        pltpu.sync_copy(rows_v, out_hbm.at[pl.ds(base, b_per_w)])

    return gather(codebook_t, idx)


def _finalize_body(x_ref, q_ref, idx_ref, ste_ref, ppl_ref, cb_ref, cm_ref,
                   loss_s, cnt_s):
    i = pl.program_id(0)
    ni = pl.num_programs(0)
    x = x_ref[...]
    q = q_ref[...]
    diff = q - x
    ste_ref[...] = x + diff
    sq = jnp.sum(diff * diff)

    @pl.when(i == 0)
    def _():
        loss_s[0] = sq
        cnt_s[...] = jnp.zeros((K_CODES,), jnp.float32)

    @pl.when(i > 0)
    def _():
        loss_s[0] = loss_s[0] + sq

    idx = idx_ref[...]
    nb = K_CODES // BBIN
    for jb in range(nb):
        bins = jb * BBIN + lax.broadcasted_iota(jnp.int32, (1, BBIN), 1)
        hits = (idx[:, None] == bins).astype(jnp.float32)
        cnt_s[pl.ds(jb * BBIN, BBIN)] = (
            cnt_s[pl.ds(jb * BBIN, BBIN)] + jnp.sum(hits, axis=0))

    @pl.when(i == ni - 1)
    def _():
        mse = loss_s[0] / jnp.float32(N_ROWS * DIM)
        cb_ref[...] = mse[None, None]
        cm_ref[...] = (COMMIT * mse)[None, None]
        p = cnt_s[...] * jnp.float32(1.0 / N_ROWS)
        entropy = -jnp.sum(p * jnp.log(p + 1e-10))
        ppl_ref[...] = jnp.exp(entropy)[None, None]


def _finalize(flat_x, quantized, idx):
    grid = (N_ROWS // BM3,)
    scalar = jax.ShapeDtypeStruct((1, 1), jnp.float32)
    return pl.pallas_call(
        _finalize_body,
        grid=grid,
        in_specs=[
            pl.BlockSpec((BM3, DIM), lambda i: (i, 0)),
            pl.BlockSpec((BM3, DIM), lambda i: (i, 0)),
            pl.BlockSpec((BM3,), lambda i: (i,)),
        ],
        out_specs=[
            pl.BlockSpec((BM3, DIM), lambda i: (i, 0)),
            pl.BlockSpec((1, 1), lambda i: (0, 0)),
            pl.BlockSpec((1, 1), lambda i: (0, 0)),
            pl.BlockSpec((1, 1), lambda i: (0, 0)),
        ],
        out_shape=[
            jax.ShapeDtypeStruct((N_ROWS, DIM), jnp.float32),
            scalar, scalar, scalar,
        ],
        scratch_shapes=[
            pltpu.SMEM((1,), jnp.float32),
            pltpu.VMEM((K_CODES,), jnp.float32),
        ],
    )(flat_x, quantized, idx)


def kernel(inputs, codebook):
    flat_x = inputs.reshape(-1, DIM)
    idx = _argmin_indices(flat_x, codebook)
    quantized = _gather_quantized(codebook.T, idx)
    ste, ppl, cb_loss, cm_loss = _finalize(flat_x, quantized, idx)
    return (
        ste.reshape(inputs.shape),
        ppl.reshape(()),
        cb_loss.reshape(()),
        cm_loss.reshape(()),
        idx.reshape(inputs.shape[0], -1),
    )


# speedup vs baseline: 1.1151x; 1.0566x over previous
import functools

import jax
import jax.numpy as jnp
from jax import lax
from jax.experimental import pallas as pl
from jax.experimental.pallas import tpu as pltpu
from jax.experimental.pallas import tpu_sc as plsc

N_ROWS = 8192
DIM = 256
K_CODES = 8192
COMMIT = 0.25

BM = 2048
BN = 1024
BM3 = 256
BBIN = 1024


def _argmin_body(x_ref, c_ref, o_ref):
    x = x_ref[...]
    x2 = jnp.sum(x * x, axis=1, keepdims=True)
    rmin = jnp.full((BM,), jnp.inf, jnp.float32)
    rarg = jnp.zeros((BM,), jnp.int32)
    for j in range(K_CODES // BN):
        c = c_ref[:, j * BN:(j + 1) * BN]
        c2 = jnp.sum(c * c, axis=0, keepdims=True)
        m = jnp.dot(x, c, preferred_element_type=jnp.float32)
        d = x2 + c2 - 2.0 * m
        bmin = jnp.min(d, axis=1, keepdims=True)
        iota = lax.broadcasted_iota(jnp.int32, (BM, BN), 1)
        cand = jnp.where(d == bmin, iota, BN)
        barg = jnp.min(cand, axis=1) + j * BN
        bminf = bmin[:, 0]
        better = bminf < rmin
        rarg = jnp.where(better, barg, rarg)
        rmin = jnp.where(better, bminf, rmin)
    o_ref[...] = rarg


def _argmin_indices(flat_x, codebook):
    grid = (N_ROWS // BM,)
    return pl.pallas_call(
        _argmin_body,
        grid=grid,
        in_specs=[
            pl.BlockSpec((BM, DIM), lambda i: (i, 0)),
            pl.BlockSpec((DIM, K_CODES), lambda i: (0, 0)),
        ],
        out_specs=pl.BlockSpec((BM,), lambda i: (i,)),
        out_shape=jax.ShapeDtypeStruct((N_ROWS,), jnp.int32),
    )(flat_x, codebook)


def _gather_quantized(codebook_t, idx):
    info = plsc.get_sparse_core_info()
    nw = info.num_cores * info.num_subcores
    b_per_w = N_ROWS // nw

    mesh = plsc.VectorSubcoreMesh(core_axis_name="c", subcore_axis_name="s")

    @functools.partial(
        pl.kernel,
        mesh=mesh,
        out_type=jax.ShapeDtypeStruct((N_ROWS, DIM), jnp.float32),
        scratch_types=[
            pltpu.VMEM((b_per_w,), jnp.int32),
            pltpu.VMEM((b_per_w, DIM), jnp.float32),
            pltpu.SemaphoreType.DMA,
        ],
    )
    def gather(table_hbm, idx_hbm, out_hbm, idx_v, rows_v, sem):
        wid = lax.axis_index("s") * info.num_cores + lax.axis_index("c")
        base = wid * b_per_w
        pltpu.sync_copy(idx_hbm.at[pl.ds(base, b_per_w)], idx_v)
        pltpu.async_copy(table_hbm.at[idx_v], rows_v, sem).wait()
        pltpu.sync_copy(rows_v, out_hbm.at[pl.ds(base, b_per_w)])

    return gather(codebook_t, idx)


def _finalize_body(x_ref, q_ref, idx_ref, ste_ref, ppl_ref, cb_ref, cm_ref,
                   loss_s, cnt_s):
    i = pl.program_id(0)
    ni = pl.num_programs(0)
    x = x_ref[...]
    q = q_ref[...]
    diff = q - x
    ste_ref[...] = x + diff
    sq = jnp.sum(diff * diff)

    @pl.when(i == 0)
    def _():
        loss_s[0] = sq
        cnt_s[...] = jnp.zeros((K_CODES,), jnp.float32)

    @pl.when(i > 0)
    def _():
        loss_s[0] = loss_s[0] + sq

    idx = idx_ref[...]
    nb = K_CODES // BBIN
    for jb in range(nb):
        bins = jb * BBIN + lax.broadcasted_iota(jnp.int32, (1, BBIN), 1)
        hits = (idx[:, None] == bins).astype(jnp.float32)
        cnt_s[pl.ds(jb * BBIN, BBIN)] = (
            cnt_s[pl.ds(jb * BBIN, BBIN)] + jnp.sum(hits, axis=0))

    @pl.when(i == ni - 1)
    def _():
        mse = loss_s[0] / jnp.float32(N_ROWS * DIM)
        cb_ref[...] = mse[None, None]
        cm_ref[...] = (COMMIT * mse)[None, None]
        p = cnt_s[...] * jnp.float32(1.0 / N_ROWS)
        entropy = -jnp.sum(p * jnp.log(p + 1e-10))
        ppl_ref[...] = jnp.exp(entropy)[None, None]


def _finalize(flat_x, quantized, idx):
    grid = (N_ROWS // BM3,)
    scalar = jax.ShapeDtypeStruct((1, 1), jnp.float32)
    return pl.pallas_call(
        _finalize_body,
        grid=grid,
        in_specs=[
            pl.BlockSpec((BM3, DIM), lambda i: (i, 0)),
            pl.BlockSpec((BM3, DIM), lambda i: (i, 0)),
            pl.BlockSpec((BM3,), lambda i: (i,)),
        ],
        out_specs=[
            pl.BlockSpec((BM3, DIM), lambda i: (i, 0)),
            pl.BlockSpec((1, 1), lambda i: (0, 0)),
            pl.BlockSpec((1, 1), lambda i: (0, 0)),
            pl.BlockSpec((1, 1), lambda i: (0, 0)),
        ],
        out_shape=[
            jax.ShapeDtypeStruct((N_ROWS, DIM), jnp.float32),
            scalar, scalar, scalar,
        ],
        scratch_shapes=[
            pltpu.SMEM((1,), jnp.float32),
            pltpu.VMEM((K_CODES,), jnp.float32),
        ],
    )(flat_x, quantized, idx)


def kernel(inputs, codebook):
    flat_x = inputs.reshape(-1, DIM)
    idx = _argmin_indices(flat_x, codebook)
    quantized = _gather_quantized(codebook.T, idx)
    ste, ppl, cb_loss, cm_loss = _finalize(flat_x, quantized, idx)
    return (
        ste.reshape(inputs.shape),
        ppl.reshape(()),
        cb_loss.reshape(()),
        cm_loss.reshape(()),
        idx.reshape(inputs.shape[0], -1),
    )


# speedup vs baseline: 1.1541x; 1.0350x over previous
import functools

import jax
import jax.numpy as jnp
from jax import lax
from jax.experimental import pallas as pl
from jax.experimental.pallas import tpu as pltpu
from jax.experimental.pallas import tpu_sc as plsc

N_ROWS = 8192
DIM = 256
K_CODES = 8192
COMMIT = 0.25

BM = 2048
BN = 2048
BM3 = 256
BBIN = 1024


def _argmin_body(x_ref, c_ref, o_ref):
    x = x_ref[...]
    x2 = jnp.sum(x * x, axis=1, keepdims=True)
    rmin = jnp.full((BM,), jnp.inf, jnp.float32)
    rarg = jnp.zeros((BM,), jnp.int32)
    for j in range(K_CODES // BN):
        c = c_ref[:, j * BN:(j + 1) * BN]
        c2 = jnp.sum(c * c, axis=0, keepdims=True)
        m = jnp.dot(x, c, preferred_element_type=jnp.float32)
        d = x2 + c2 - 2.0 * m
        bmin = jnp.min(d, axis=1, keepdims=True)
        iota = lax.broadcasted_iota(jnp.int32, (BM, BN), 1)
        cand = jnp.where(d == bmin, iota, BN)
        barg = jnp.min(cand, axis=1) + j * BN
        bminf = bmin[:, 0]
        better = bminf < rmin
        rarg = jnp.where(better, barg, rarg)
        rmin = jnp.where(better, bminf, rmin)
    o_ref[...] = rarg


def _argmin_indices(flat_x, codebook):
    grid = (N_ROWS // BM,)
    return pl.pallas_call(
        _argmin_body,
        grid=grid,
        in_specs=[
            pl.BlockSpec((BM, DIM), lambda i: (i, 0)),
            pl.BlockSpec((DIM, K_CODES), lambda i: (0, 0)),
        ],
        out_specs=pl.BlockSpec((BM,), lambda i: (i,)),
        out_shape=jax.ShapeDtypeStruct((N_ROWS,), jnp.int32),
    )(flat_x, codebook)


def _gather_quantized(codebook_t, idx):
    info = plsc.get_sparse_core_info()
    nw = info.num_cores * info.num_subcores
    b_per_w = N_ROWS // nw

    mesh = plsc.VectorSubcoreMesh(core_axis_name="c", subcore_axis_name="s")

    @functools.partial(
        pl.kernel,
        mesh=mesh,
        out_type=jax.ShapeDtypeStruct((N_ROWS, DIM), jnp.float32),
        scratch_types=[
            pltpu.VMEM((b_per_w,), jnp.int32),
            pltpu.VMEM((b_per_w, DIM), jnp.float32),
            pltpu.SemaphoreType.DMA,
        ],
    )
    def gather(table_hbm, idx_hbm, out_hbm, idx_v, rows_v, sem):
        wid = lax.axis_index("s") * info.num_cores + lax.axis_index("c")
        base = wid * b_per_w
        pltpu.sync_copy(idx_hbm.at[pl.ds(base, b_per_w)], idx_v)
        pltpu.async_copy(table_hbm.at[idx_v], rows_v, sem).wait()
        pltpu.sync_copy(rows_v, out_hbm.at[pl.ds(base, b_per_w)])

    return gather(codebook_t, idx)


def _finalize_body(x_ref, q_ref, idx_ref, ste_ref, ppl_ref, cb_ref, cm_ref,
                   loss_s, cnt_s):
    i = pl.program_id(0)
    ni = pl.num_programs(0)
    x = x_ref[...]
    q = q_ref[...]
    diff = q - x
    ste_ref[...] = x + diff
    sq = jnp.sum(diff * diff)

    @pl.when(i == 0)
    def _():
        loss_s[0] = sq
        cnt_s[...] = jnp.zeros((K_CODES,), jnp.float32)

    @pl.when(i > 0)
    def _():
        loss_s[0] = loss_s[0] + sq

    idx = idx_ref[...]
    nb = K_CODES // BBIN
    for jb in range(nb):
        bins = jb * BBIN + lax.broadcasted_iota(jnp.int32, (1, BBIN), 1)
        hits = (idx[:, None] == bins).astype(jnp.float32)
        cnt_s[pl.ds(jb * BBIN, BBIN)] = (
            cnt_s[pl.ds(jb * BBIN, BBIN)] + jnp.sum(hits, axis=0))

    @pl.when(i == ni - 1)
    def _():
        mse = loss_s[0] / jnp.float32(N_ROWS * DIM)
        cb_ref[...] = mse[None, None]
        cm_ref[...] = (COMMIT * mse)[None, None]
        p = cnt_s[...] * jnp.float32(1.0 / N_ROWS)
        entropy = -jnp.sum(p * jnp.log(p + 1e-10))
        ppl_ref[...] = jnp.exp(entropy)[None, None]


def _finalize(flat_x, quantized, idx):
    grid = (N_ROWS // BM3,)
    scalar = jax.ShapeDtypeStruct((1, 1), jnp.float32)
    return pl.pallas_call(
        _finalize_body,
        grid=grid,
        in_specs=[
            pl.BlockSpec((BM3, DIM), lambda i: (i, 0)),
            pl.BlockSpec((BM3, DIM), lambda i: (i, 0)),
            pl.BlockSpec((BM3,), lambda i: (i,)),
        ],
        out_specs=[
            pl.BlockSpec((BM3, DIM), lambda i: (i, 0)),
            pl.BlockSpec((1, 1), lambda i: (0, 0)),
            pl.BlockSpec((1, 1), lambda i: (0, 0)),
            pl.BlockSpec((1, 1), lambda i: (0, 0)),
        ],
        out_shape=[
            jax.ShapeDtypeStruct((N_ROWS, DIM), jnp.float32),
            scalar, scalar, scalar,
        ],
        scratch_shapes=[
            pltpu.SMEM((1,), jnp.float32),
            pltpu.VMEM((K_CODES,), jnp.float32),
        ],
    )(flat_x, quantized, idx)


def kernel(inputs, codebook):
    flat_x = inputs.reshape(-1, DIM)
    idx = _argmin_indices(flat_x, codebook)
    quantized = _gather_quantized(codebook.T, idx)
    ste, ppl, cb_loss, cm_loss = _finalize(flat_x, quantized, idx)
    return (
        ste.reshape(inputs.shape),
        ppl.reshape(()),
        cb_loss.reshape(()),
        cm_loss.reshape(()),
        idx.reshape(inputs.shape[0], -1),
    )
